# baseline (speedup 1.0000x reference)
import functools

import jax
import jax.numpy as jnp
from jax import lax
from jax.experimental import pallas as pl
from jax.experimental.pallas import tpu as pltpu
from jax.experimental.pallas import tpu_sc as plsc

_B = 16384
_F = 26
_V = 2
_D = 128

_NC = 2
_NS = 16
_NW = _NC * _NS

_REPS = 256
_N = _B * _F
_PER_W = _N // _NW
_CH = 128
_NCH = _PER_W // _CH


def _lookup(xf, table2):
    mesh = plsc.VectorSubcoreMesh(core_axis_name="c", subcore_axis_name="s")

    @functools.partial(
        pl.kernel,
        out_type=jax.ShapeDtypeStruct((_N, _D), jnp.float32),
        mesh=mesh,
        scratch_types=[
            pltpu.VMEM((_PER_W,), jnp.int32),
            pltpu.VMEM((2, _CH), jnp.int32),
            pltpu.VMEM((2, _CH, _D), jnp.float32),
            pltpu.SemaphoreType.DMA,
            pltpu.SemaphoreType.DMA,
            pltpu.SemaphoreType.DMA,
            pltpu.SemaphoreType.DMA,
        ],
    )
    def body(xf_hbm, tab_hbm, out_hbm, xall, idx2, rows2, g0, g1, s0, s1):
        gsem = (g0, g1)
        osem = (s0, s1)
        wid = lax.axis_index("s") * _NC + lax.axis_index("c")
        wbase = wid * _PER_W
        pltpu.sync_copy(xf_hbm.at[pl.ds(wbase, _PER_W)], xall)

        lanes = lax.iota(jnp.int32, 16)

        def compute_idx(j, b):
            base = j * _CH
            for g in range(_CH // 16):
                off = base + g * 16
                pos = (wbase + off) + lanes
                f = lax.rem(pos, _F)
                rep = lax.bitwise_and(pos, _REPS - 1) * (_F * _V)
                idx2[b, pl.ds(g * 16, 16)] = (
                    xall[pl.ds(off, 16)] + 2 * f + rep)

        def fire_gather(b):
            pltpu.async_copy(tab_hbm.at[idx2.at[b]], rows2.at[b], gsem[b])

        def wait_gather(b):
            pltpu.make_async_copy(
                tab_hbm.at[idx2.at[b]], rows2.at[b], gsem[b]).wait()

        def fire_scatter(j, b):
            pltpu.async_copy(
                rows2.at[b], out_hbm.at[pl.ds(wbase + j * _CH, _CH)], osem[b])

        def wait_scatter(b):
            pltpu.make_async_copy(
                rows2.at[b], out_hbm.at[pl.ds(wbase, _CH)], osem[b]).wait()

        compute_idx(0, 0)
        fire_gather(0)
        wait_gather(0)
        fire_scatter(0, 0)
        compute_idx(1, 1)
        fire_gather(1)

        def outer(s, carry):
            for k in range(2):
                b = (1 + k) % 2
                j = 1 + s * 2 + k
                wait_gather(b)
                fire_scatter(j, b)
                wait_scatter(1 - b)
                compute_idx(j + 1, 1 - b)
                fire_gather(1 - b)
            return carry

        lax.fori_loop(0, (_NCH - 2) // 2, outer, 0)

        wait_gather(1)
        fire_scatter(_NCH - 1, 1)
        wait_scatter(0)
        wait_scatter(1)

    return body(xf, table2)


def kernel(X, tables):
    xf = X.reshape(_N)
    table2 = jnp.tile(tables.reshape(_F * _V, _D), (_REPS, 1))
    out = _lookup(xf, table2)
    return out.reshape(_B, _F, 1, _D)


# speedup vs baseline: 1.1238x; 1.1238x over previous
import functools

import jax
import jax.numpy as jnp
from jax import lax
from jax.experimental import pallas as pl
from jax.experimental.pallas import tpu as pltpu
from jax.experimental.pallas import tpu_sc as plsc

_B = 16384
_F = 26
_V = 2
_D = 128

_NC = 2
_NS = 16
_NW = _NC * _NS

_REPS = 256
_N = _B * _F
_PER_W = _N // _NW
_CH = 128
_NCH = _PER_W // _CH
_NBUF = 4


def _lookup(xf, table2):
    mesh = plsc.VectorSubcoreMesh(core_axis_name="c", subcore_axis_name="s")

    @functools.partial(
        pl.kernel,
        out_type=jax.ShapeDtypeStruct((_N, _D), jnp.float32),
        mesh=mesh,
        scratch_types=[
            pltpu.VMEM((_PER_W,), jnp.int32),
            pltpu.VMEM((_NBUF, _CH), jnp.int32),
            pltpu.VMEM((_NBUF, _CH, _D), jnp.float32),
            pltpu.SemaphoreType.DMA,
            pltpu.SemaphoreType.DMA,
            pltpu.SemaphoreType.DMA,
            pltpu.SemaphoreType.DMA,
            pltpu.SemaphoreType.DMA,
            pltpu.SemaphoreType.DMA,
            pltpu.SemaphoreType.DMA,
            pltpu.SemaphoreType.DMA,
        ],
    )
    def body(xf_hbm, tab_hbm, out_hbm, xall, idxs, rows,
             g0, g1, g2, g3, s0, s1, s2, s3):
        gsem = (g0, g1, g2, g3)
        osem = (s0, s1, s2, s3)
        wid = lax.axis_index("s") * _NC + lax.axis_index("c")
        wbase = wid * _PER_W
        pltpu.sync_copy(xf_hbm.at[pl.ds(wbase, _PER_W)], xall)

        lanes = lax.iota(jnp.int32, 16)

        def compute_idx(j, b):
            base = j * _CH
            for g in range(_CH // 16):
                off = base + g * 16
                pos = (wbase + off) + lanes
                f = lax.rem(pos, _F)
                rep = lax.bitwise_and(pos, _REPS - 1) * (_F * _V)
                idxs[b, pl.ds(g * 16, 16)] = (
                    xall[pl.ds(off, 16)] + 2 * f + rep)

        def fire_gather(b):
            pltpu.async_copy(tab_hbm.at[idxs.at[b]], rows.at[b], gsem[b])

        def wait_gather(b):
            pltpu.make_async_copy(
                tab_hbm.at[idxs.at[b]], rows.at[b], gsem[b]).wait()

        def fire_scatter(j, b):
            pltpu.async_copy(
                rows.at[b], out_hbm.at[pl.ds(wbase + j * _CH, _CH)], osem[b])

        def wait_scatter(b):
            pltpu.make_async_copy(
                rows.at[b], out_hbm.at[pl.ds(wbase, _CH)], osem[b]).wait()

        for j in range(2):
            compute_idx(j, j)
            fire_gather(j)
        for j in range(2):
            wait_gather(j)
            fire_scatter(j, j)
            compute_idx(j + 2, j + 2)
            fire_gather(j + 2)

        def outer(s, carry):
            for k in range(_NBUF):
                j = 2 + s * _NBUF + k
                b = (2 + k) % _NBUF
                bg = (k) % _NBUF
                wait_gather(b)
                fire_scatter(j, b)
                wait_scatter(bg)
                compute_idx(j + 2, bg)
                fire_gather(bg)
            return carry

        lax.fori_loop(0, (_NCH - 4) // _NBUF, outer, 0)

        for j in range(_NCH - 2, _NCH):
            b = j % _NBUF
            wait_gather(b)
            fire_scatter(j, b)
        for b in range(_NBUF):
            wait_scatter(b)

    return body(xf, table2)


def kernel(X, tables):
    xf = X.reshape(_N)
    table2 = jnp.tile(tables.reshape(_F * _V, _D), (_REPS, 1))
    out = _lookup(xf, table2)
    return out.reshape(_B, _F, 1, _D)
